# Initial kernel scaffold; baseline (speedup 1.0000x reference)
import jax
import jax.numpy as jnp
from jax.experimental import pallas as pl


def kernel(nodes_feat, edge_index, edges_feat, nodes_num_norm_sqrt, edges_num_norm_sqrt, W0, b0, gamma0, beta0, W1, b1, gamma1, beta1):
    raise NotImplementedError("write your pallas kernel here")


# speedup vs baseline: 3.4605x; 3.4605x over previous
import functools

import jax
import jax.numpy as jnp
from jax import lax
from jax.experimental import pallas as pl
from jax.experimental.pallas import tpu as pltpu
from jax.experimental.pallas import tpu_sc as plsc

N = 10000
E = 320000
D = 128
NC, NS, L = 2, 16, 16
NW = NC * NS
N_PAD = 10240
E_PER_W = E // NW + 240
E_PAD = NW * E_PER_W
CHUNK = 128
CH_PER_W = E_PER_W // CHUNK
ROWS_PER_TILE = N_PAD // NS

_mesh = plsc.VectorSubcoreMesh(
    core_axis_name="c", subcore_axis_name="s", num_cores=NC, num_subcores=NS)
_sc_params = pltpu.CompilerParams(needs_layout_passes=False)


def _degree_body(src_hbm, dst_hbm, degs_hbm, degd_hbm, idx_s, idx_d, hist_s, hist_d):
    c = lax.axis_index("c")
    s = lax.axis_index("s")
    wid = c * NS + s
    ones = jnp.ones((L,), jnp.float32)

    def zero(i, _):
        hist_s[pl.ds(i * L, L)] = jnp.zeros((L,), jnp.float32)
        hist_d[pl.ds(i * L, L)] = jnp.zeros((L,), jnp.float32)
        return 0
    lax.fori_loop(0, N_PAD // L, zero, 0, unroll=4)

    pltpu.sync_copy(src_hbm.at[pl.ds(wid * E_PER_W, E_PER_W)], idx_s)
    pltpu.sync_copy(dst_hbm.at[pl.ds(wid * E_PER_W, E_PER_W)], idx_d)

    def hist(j, _):
        vs = idx_s[pl.ds(j * L, L)]
        plsc.addupdate_scatter(hist_s, [vs], ones)
        vd = idx_d[pl.ds(j * L, L)]
        plsc.addupdate_scatter(hist_d, [vd], ones)
        return 0
    lax.fori_loop(0, E_PER_W // L, hist, 0, unroll=4)

    pltpu.sync_copy(hist_s, degs_hbm.at[wid])
    pltpu.sync_copy(hist_d, degd_hbm.at[wid])


_degree_call = pl.kernel(
    _degree_body,
    out_type=[jax.ShapeDtypeStruct((NW, N_PAD), jnp.float32),
              jax.ShapeDtypeStruct((NW, N_PAD), jnp.float32)],
    mesh=_mesh,
    scratch_types=[pltpu.VMEM((E_PER_W,), jnp.int32),
                   pltpu.VMEM((E_PER_W,), jnp.int32),
                   pltpu.VMEM((N_PAD,), jnp.float32),
                   pltpu.VMEM((N_PAD,), jnp.float32)],
    compiler_params=_sc_params,
)


def _scatter_body(xs_hbm, src2_hbm, dst2_hbm, part_hbm,
                  idx_s, idx_d, rowbuf, zbuf, shared, gsem):
    c = lax.axis_index("c")
    s = lax.axis_index("s")
    wid = c * NS + s

    for r in range(L):
        for l in range(D // L):
            zbuf[r, pl.ds(l * L, L)] = jnp.zeros((L,), jnp.float32)

    def zero(i, _):
        pltpu.sync_copy(zbuf, shared.at[pl.ds(s * ROWS_PER_TILE + i * L, L)])
        return 0
    lax.fori_loop(0, ROWS_PER_TILE // L, zero, 0)
    plsc.subcore_barrier()

    pltpu.sync_copy(src2_hbm.at[pl.ds(wid * CH_PER_W, CH_PER_W)], idx_s)
    pltpu.sync_copy(dst2_hbm.at[pl.ds(wid * CH_PER_W, CH_PER_W)], idx_d)

    def edges(ch, _):
        pltpu.async_copy(xs_hbm.at[idx_s.at[ch]], rowbuf, gsem).wait()
        pltpu.sync_copy(rowbuf, shared.at[idx_d.at[ch]], add=True)
        return 0
    lax.fori_loop(0, CH_PER_W, edges, 0)
    plsc.subcore_barrier()

    pltpu.sync_copy(shared.at[pl.ds(s * ROWS_PER_TILE, ROWS_PER_TILE)],
                    part_hbm.at[c].at[pl.ds(s * ROWS_PER_TILE, ROWS_PER_TILE)])


_scatter_call = pl.kernel(
    _scatter_body,
    out_type=jax.ShapeDtypeStruct((NC, N_PAD, D), jnp.float32),
    mesh=_mesh,
    scratch_types=[pltpu.VMEM((CH_PER_W, CHUNK), jnp.int32),
                   pltpu.VMEM((CH_PER_W, CHUNK), jnp.int32),
                   pltpu.VMEM((CHUNK, D), jnp.float32),
                   pltpu.VMEM((L, D), jnp.float32),
                   pltpu.VMEM_SHARED((N_PAD, D), jnp.float32),
                   pltpu.SemaphoreType.DMA],
    compiler_params=_sc_params,
)


def _norms_body(hs_ref, hd_ref, ns_ref, nd_ref):
    ds_ = jnp.sum(hs_ref[...], axis=0)
    dd = jnp.sum(hd_ref[...], axis=0)
    ns_ref[...] = jnp.where(ds_ > 0, lax.rsqrt(jnp.maximum(ds_, 1.0)), 0.0)
    nd_ref[...] = jnp.where(dd > 0, lax.rsqrt(jnp.maximum(dd, 1.0)), 0.0)


_norms_call = pl.pallas_call(
    _norms_body,
    out_shape=[jax.ShapeDtypeStruct((N_PAD,), jnp.float32),
               jax.ShapeDtypeStruct((N_PAD,), jnp.float32)],
)


def _pre_body(h_ref, w_ref, ns_ref, xs_ref):
    x = jnp.dot(h_ref[...], w_ref[...], preferred_element_type=jnp.float32)
    xs_ref[0:N, :] = x * ns_ref[0:N, :]
    xs_ref[N:, :] = jnp.zeros((N_PAD - N, D), jnp.float32)


_pre_call = pl.pallas_call(
    _pre_body,
    out_shape=jax.ShapeDtypeStruct((N_PAD, D), jnp.float32),
)


def _post(p_ref, nd_ref, sn_ref, b_ref, g_ref, be_ref, h_prev):
    agg = p_ref[0, 0:N, :] + p_ref[1, 0:N, :]
    x = agg * nd_ref[0:N, :] + b_ref[...]
    x = x * sn_ref[...]
    mean = jnp.mean(x, axis=0)
    var = jnp.mean((x - mean) ** 2, axis=0)
    x = (x - mean) * lax.rsqrt(var + 1e-5) * g_ref[...] + be_ref[...]
    return h_prev + jnp.maximum(x, 0.0)


def _mid_body(p_ref, nd_ref, sn_ref, b_ref, g_ref, be_ref, h0_ref, ns_ref,
              w1_ref, xs1_ref, h1_ref):
    h1 = _post(p_ref, nd_ref, sn_ref, b_ref, g_ref, be_ref, h0_ref[...])
    h1_ref[...] = h1
    x1 = jnp.dot(h1, w1_ref[...], preferred_element_type=jnp.float32)
    xs1_ref[0:N, :] = x1 * ns_ref[0:N, :]
    xs1_ref[N:, :] = jnp.zeros((N_PAD - N, D), jnp.float32)


_mid_call = pl.pallas_call(
    _mid_body,
    out_shape=[jax.ShapeDtypeStruct((N_PAD, D), jnp.float32),
               jax.ShapeDtypeStruct((N, D), jnp.float32)],
)


def _final_body(p_ref, nd_ref, sn_ref, b_ref, g_ref, be_ref, h1_ref, hg_ref):
    h2 = _post(p_ref, nd_ref, sn_ref, b_ref, g_ref, be_ref, h1_ref[...])
    hg_ref[...] = jnp.mean(h2, axis=0, keepdims=True)


_final_call = pl.pallas_call(
    _final_body,
    out_shape=jax.ShapeDtypeStruct((1, D), jnp.float32),
)


def kernel(nodes_feat, edge_index, edges_feat, nodes_num_norm_sqrt,
           edges_num_norm_sqrt, W0, b0, gamma0, beta0, W1, b1, gamma1, beta1):
    del edges_feat, edges_num_norm_sqrt
    src = edge_index[0]
    dst = edge_index[1]
    padv = jnp.full((E_PAD - E,), N, jnp.int32)
    src1 = jnp.concatenate([src, padv])
    dst1 = jnp.concatenate([dst, padv])
    src2 = src1.reshape(NW * CH_PER_W, CHUNK)
    dst2 = dst1.reshape(NW * CH_PER_W, CHUNK)

    degs, degd = _degree_call(src1, dst1)
    ns, nd = _norms_call(degs, degd)
    ns_col = ns.reshape(N_PAD, 1)
    nd_col = nd.reshape(N_PAD, 1)

    xs0 = _pre_call(nodes_feat, W0, ns_col)
    part0 = _scatter_call(xs0, src2, dst2)
    xs1, h1 = _mid_call(part0, nd_col, nodes_num_norm_sqrt, b0, gamma0, beta0,
                        nodes_feat, ns_col, W1)
    part1 = _scatter_call(xs1, src2, dst2)
    return _final_call(part1, nd_col, nodes_num_norm_sqrt, b1, gamma1, beta1, h1)


# speedup vs baseline: 3.7754x; 1.0910x over previous
import functools

import jax
import jax.numpy as jnp
from jax import lax
from jax.experimental import pallas as pl
from jax.experimental.pallas import tpu as pltpu
from jax.experimental.pallas import tpu_sc as plsc

N = 10000
E = 320000
D = 128
NC, NS, L = 2, 16, 16
NW = NC * NS
N_PAD = 10240
E_PER_W = E // NW + 240
E_PAD = NW * E_PER_W
CHUNK = 128
CH_PER_W = E_PER_W // CHUNK
CH_GRP = 16
ROWS_PER_TILE = N_PAD // NS

_mesh = plsc.VectorSubcoreMesh(
    core_axis_name="c", subcore_axis_name="s", num_cores=NC, num_subcores=NS)
_sc_params = pltpu.CompilerParams(needs_layout_passes=False)


def _degree_body(src_hbm, dst_hbm, degs_hbm, degd_hbm, idx_s, idx_d, hist_s, hist_d):
    c = lax.axis_index("c")
    s = lax.axis_index("s")
    wid = c * NS + s
    ones = jnp.ones((L,), jnp.float32)

    def zero(i, _):
        hist_s[pl.ds(i * L, L)] = jnp.zeros((L,), jnp.float32)
        hist_d[pl.ds(i * L, L)] = jnp.zeros((L,), jnp.float32)
        return 0
    lax.fori_loop(0, N_PAD // L, zero, 0, unroll=4)

    pltpu.sync_copy(src_hbm.at[pl.ds(wid * E_PER_W, E_PER_W)], idx_s)
    pltpu.sync_copy(dst_hbm.at[pl.ds(wid * E_PER_W, E_PER_W)], idx_d)

    def hist(j, _):
        vs = idx_s[pl.ds(j * L, L)]
        plsc.addupdate_scatter(hist_s, [vs], ones)
        vd = idx_d[pl.ds(j * L, L)]
        plsc.addupdate_scatter(hist_d, [vd], ones)
        return 0
    lax.fori_loop(0, E_PER_W // L, hist, 0, unroll=4)

    pltpu.sync_copy(hist_s, degs_hbm.at[wid])
    pltpu.sync_copy(hist_d, degd_hbm.at[wid])


_degree_call = pl.kernel(
    _degree_body,
    out_type=[jax.ShapeDtypeStruct((NW, N_PAD), jnp.float32),
              jax.ShapeDtypeStruct((NW, N_PAD), jnp.float32)],
    mesh=_mesh,
    scratch_types=[pltpu.VMEM((E_PER_W,), jnp.int32),
                   pltpu.VMEM((E_PER_W,), jnp.int32),
                   pltpu.VMEM((N_PAD,), jnp.float32),
                   pltpu.VMEM((N_PAD,), jnp.float32)],
    compiler_params=_sc_params,
)


def _scatter_body(xs_hbm, src2_hbm, dst2_hbm, part_hbm,
                  idx_s, idx_d, buf0, buf1, zbuf, shared, sem0, sem1):
    c = lax.axis_index("c")
    s = lax.axis_index("s")
    wid = c * NS + s

    for r in range(L):
        for l in range(D // L):
            zbuf[r, pl.ds(l * L, L)] = jnp.zeros((L,), jnp.float32)

    def zero(i, _):
        pltpu.sync_copy(zbuf, shared.at[pl.ds(s * ROWS_PER_TILE + i * L, L)])
        return 0
    lax.fori_loop(0, ROWS_PER_TILE // L, zero, 0)
    plsc.subcore_barrier()

    npair = CH_GRP // 2

    def group(g, _):
        gbase = wid * CH_PER_W + g * CH_GRP
        pltpu.sync_copy(src2_hbm.at[pl.ds(gbase, CH_GRP)], idx_s)
        pltpu.sync_copy(dst2_hbm.at[pl.ds(gbase, CH_GRP)], idx_d)
        pltpu.async_copy(xs_hbm.at[idx_s.at[0]], buf0, sem0)

        def edges(j, _):
            pltpu.async_copy(xs_hbm.at[idx_s.at[2 * j + 1]], buf1, sem1)
            pltpu.make_async_copy(xs_hbm.at[pl.ds(0, CHUNK)], buf0, sem0).wait()
            pltpu.sync_copy(buf0, shared.at[idx_d.at[2 * j]], add=True)

            @pl.when(j < npair - 1)
            def _():
                pltpu.async_copy(xs_hbm.at[idx_s.at[2 * j + 2]], buf0, sem0)
            pltpu.make_async_copy(xs_hbm.at[pl.ds(0, CHUNK)], buf1, sem1).wait()
            pltpu.sync_copy(buf1, shared.at[idx_d.at[2 * j + 1]], add=True)
            return 0
        lax.fori_loop(0, npair, edges, 0)
        return 0
    lax.fori_loop(0, CH_PER_W // CH_GRP, group, 0)
    plsc.subcore_barrier()

    pltpu.sync_copy(shared.at[pl.ds(s * ROWS_PER_TILE, ROWS_PER_TILE)],
                    part_hbm.at[c].at[pl.ds(s * ROWS_PER_TILE, ROWS_PER_TILE)])


_scatter_call = pl.kernel(
    _scatter_body,
    out_type=jax.ShapeDtypeStruct((NC, N_PAD, D), jnp.float32),
    mesh=_mesh,
    scratch_types=[pltpu.VMEM((CH_GRP, CHUNK), jnp.int32),
                   pltpu.VMEM((CH_GRP, CHUNK), jnp.int32),
                   pltpu.VMEM((CHUNK, D), jnp.float32),
                   pltpu.VMEM((CHUNK, D), jnp.float32),
                   pltpu.VMEM((L, D), jnp.float32),
                   pltpu.VMEM_SHARED((N_PAD, D), jnp.float32),
                   pltpu.SemaphoreType.DMA,
                   pltpu.SemaphoreType.DMA],
    compiler_params=_sc_params,
)


def _norms_body(hs_ref, hd_ref, ns_ref, nd_ref):
    ds_ = jnp.sum(hs_ref[...], axis=0)
    dd = jnp.sum(hd_ref[...], axis=0)
    ns_ref[...] = jnp.where(ds_ > 0, lax.rsqrt(jnp.maximum(ds_, 1.0)), 0.0)
    nd_ref[...] = jnp.where(dd > 0, lax.rsqrt(jnp.maximum(dd, 1.0)), 0.0)


_norms_call = pl.pallas_call(
    _norms_body,
    out_shape=[jax.ShapeDtypeStruct((N_PAD,), jnp.float32),
               jax.ShapeDtypeStruct((N_PAD,), jnp.float32)],
)


def _pre_body(h_ref, w_ref, ns_ref, xs_ref):
    x = jnp.dot(h_ref[...], w_ref[...], preferred_element_type=jnp.float32)
    xs_ref[0:N, :] = x * ns_ref[0:N, :]
    xs_ref[N:, :] = jnp.zeros((N_PAD - N, D), jnp.float32)


_pre_call = pl.pallas_call(
    _pre_body,
    out_shape=jax.ShapeDtypeStruct((N_PAD, D), jnp.float32),
)


def _post(p_ref, nd_ref, sn_ref, b_ref, g_ref, be_ref, h_prev):
    agg = p_ref[0, 0:N, :] + p_ref[1, 0:N, :]
    x = agg * nd_ref[0:N, :] + b_ref[...]
    x = x * sn_ref[...]
    mean = jnp.mean(x, axis=0)
    var = jnp.mean((x - mean) ** 2, axis=0)
    x = (x - mean) * lax.rsqrt(var + 1e-5) * g_ref[...] + be_ref[...]
    return h_prev + jnp.maximum(x, 0.0)


def _mid_body(p_ref, nd_ref, sn_ref, b_ref, g_ref, be_ref, h0_ref, ns_ref,
              w1_ref, xs1_ref, h1_ref):
    h1 = _post(p_ref, nd_ref, sn_ref, b_ref, g_ref, be_ref, h0_ref[...])
    h1_ref[...] = h1
    x1 = jnp.dot(h1, w1_ref[...], preferred_element_type=jnp.float32)
    xs1_ref[0:N, :] = x1 * ns_ref[0:N, :]
    xs1_ref[N:, :] = jnp.zeros((N_PAD - N, D), jnp.float32)


_mid_call = pl.pallas_call(
    _mid_body,
    out_shape=[jax.ShapeDtypeStruct((N_PAD, D), jnp.float32),
               jax.ShapeDtypeStruct((N, D), jnp.float32)],
)


def _final_body(p_ref, nd_ref, sn_ref, b_ref, g_ref, be_ref, h1_ref, hg_ref):
    h2 = _post(p_ref, nd_ref, sn_ref, b_ref, g_ref, be_ref, h1_ref[...])
    hg_ref[...] = jnp.mean(h2, axis=0, keepdims=True)


_final_call = pl.pallas_call(
    _final_body,
    out_shape=jax.ShapeDtypeStruct((1, D), jnp.float32),
)


def kernel(nodes_feat, edge_index, edges_feat, nodes_num_norm_sqrt,
           edges_num_norm_sqrt, W0, b0, gamma0, beta0, W1, b1, gamma1, beta1):
    del edges_feat, edges_num_norm_sqrt
    src = edge_index[0]
    dst = edge_index[1]
    padv = jnp.full((E_PAD - E,), N, jnp.int32)
    src1 = jnp.concatenate([src, padv])
    dst1 = jnp.concatenate([dst, padv])
    src2 = src1.reshape(NW * CH_PER_W, CHUNK)
    dst2 = dst1.reshape(NW * CH_PER_W, CHUNK)

    degs, degd = _degree_call(src1, dst1)
    ns, nd = _norms_call(degs, degd)
    ns_col = ns.reshape(N_PAD, 1)
    nd_col = nd.reshape(N_PAD, 1)

    xs0 = _pre_call(nodes_feat, W0, ns_col)
    part0 = _scatter_call(xs0, src2, dst2)
    xs1, h1 = _mid_call(part0, nd_col, nodes_num_norm_sqrt, b0, gamma0, beta0,
                        nodes_feat, ns_col, W1)
    part1 = _scatter_call(xs1, src2, dst2)
    return _final_call(part1, nd_col, nodes_num_norm_sqrt, b1, gamma1, beta1, h1)


# speedup vs baseline: 3.9610x; 1.0492x over previous
import functools

import jax
import jax.numpy as jnp
from jax import lax
from jax.experimental import pallas as pl
from jax.experimental.pallas import tpu as pltpu
from jax.experimental.pallas import tpu_sc as plsc

N = 10000
E = 320000
D = 128
NC, NS, L = 2, 16, 16
NW = NC * NS
N_PAD = 10240
E_PER_W = E // NW + 240
E_PAD = NW * E_PER_W
CHUNK = 128
CH_PER_W = E_PER_W // CHUNK
CH_GRP = 16
ROWS_PER_TILE = N_PAD // NS

_mesh = plsc.VectorSubcoreMesh(
    core_axis_name="c", subcore_axis_name="s", num_cores=NC, num_subcores=NS)
_sc_params = pltpu.CompilerParams(needs_layout_passes=False)


def _degree_body(src_hbm, dst_hbm, degs_hbm, degd_hbm, idx_s, idx_d, hist_s, hist_d):
    c = lax.axis_index("c")
    s = lax.axis_index("s")
    wid = c * NS + s
    ones = jnp.ones((L,), jnp.float32)

    def zero(i, _):
        hist_s[pl.ds(i * L, L)] = jnp.zeros((L,), jnp.float32)
        hist_d[pl.ds(i * L, L)] = jnp.zeros((L,), jnp.float32)
        return 0
    lax.fori_loop(0, N_PAD // L, zero, 0, unroll=4)

    pltpu.sync_copy(src_hbm.at[pl.ds(wid * E_PER_W, E_PER_W)], idx_s)
    pltpu.sync_copy(dst_hbm.at[pl.ds(wid * E_PER_W, E_PER_W)], idx_d)

    def hist(j, _):
        vs = idx_s[pl.ds(j * L, L)]
        plsc.addupdate_scatter(hist_s, [vs], ones)
        vd = idx_d[pl.ds(j * L, L)]
        plsc.addupdate_scatter(hist_d, [vd], ones)
        return 0
    lax.fori_loop(0, E_PER_W // L, hist, 0, unroll=4)

    pltpu.sync_copy(hist_s, degs_hbm.at[wid])
    pltpu.sync_copy(hist_d, degd_hbm.at[wid])


_degree_call = pl.kernel(
    _degree_body,
    out_type=[jax.ShapeDtypeStruct((NW, N_PAD), jnp.float32),
              jax.ShapeDtypeStruct((NW, N_PAD), jnp.float32)],
    mesh=_mesh,
    scratch_types=[pltpu.VMEM((E_PER_W,), jnp.int32),
                   pltpu.VMEM((E_PER_W,), jnp.int32),
                   pltpu.VMEM((N_PAD,), jnp.float32),
                   pltpu.VMEM((N_PAD,), jnp.float32)],
    compiler_params=_sc_params,
)


def _scatter_body(xs_hbm, src2_hbm, dst2_hbm, part_hbm,
                  idx_s, idx_d, buf0, buf1, zbuf, shared, sem0, sem1):
    c = lax.axis_index("c")
    s = lax.axis_index("s")
    wid = c * NS + s

    for r in range(L):
        for l in range(D // L):
            zbuf[r, pl.ds(l * L, L)] = jnp.zeros((L,), jnp.float32)

    def zero(i, _):
        pltpu.sync_copy(zbuf, shared.at[pl.ds(s * ROWS_PER_TILE + i * L, L)])
        return 0
    lax.fori_loop(0, ROWS_PER_TILE // L, zero, 0)
    plsc.subcore_barrier()

    npair = CH_GRP // 2

    def group(g, _):
        gbase = wid * CH_PER_W + g * CH_GRP
        pltpu.sync_copy(src2_hbm.at[pl.ds(gbase, CH_GRP)], idx_s)
        pltpu.sync_copy(dst2_hbm.at[pl.ds(gbase, CH_GRP)], idx_d)
        pltpu.async_copy(xs_hbm.at[idx_s.at[0]], buf0, sem0)

        def edges(j, _):
            pltpu.async_copy(xs_hbm.at[idx_s.at[2 * j + 1]], buf1, sem1)
            pltpu.make_async_copy(xs_hbm.at[pl.ds(0, CHUNK)], buf0, sem0).wait()
            pltpu.sync_copy(buf0, shared.at[idx_d.at[2 * j]], add=True)

            @pl.when(j < npair - 1)
            def _():
                pltpu.async_copy(xs_hbm.at[idx_s.at[2 * j + 2]], buf0, sem0)
            pltpu.make_async_copy(xs_hbm.at[pl.ds(0, CHUNK)], buf1, sem1).wait()
            pltpu.sync_copy(buf1, shared.at[idx_d.at[2 * j + 1]], add=True)
            return 0
        lax.fori_loop(0, npair, edges, 0)
        return 0
    lax.fori_loop(0, CH_PER_W // CH_GRP, group, 0)
    plsc.subcore_barrier()

    pltpu.sync_copy(shared.at[pl.ds(s * ROWS_PER_TILE, ROWS_PER_TILE)],
                    part_hbm.at[c].at[pl.ds(s * ROWS_PER_TILE, ROWS_PER_TILE)])


_scatter_call = pl.kernel(
    _scatter_body,
    out_type=jax.ShapeDtypeStruct((NC, N_PAD, D), jnp.float32),
    mesh=_mesh,
    scratch_types=[pltpu.VMEM((CH_GRP, CHUNK), jnp.int32),
                   pltpu.VMEM((CH_GRP, CHUNK), jnp.int32),
                   pltpu.VMEM((CHUNK, D), jnp.float32),
                   pltpu.VMEM((CHUNK, D), jnp.float32),
                   pltpu.VMEM((L, D), jnp.float32),
                   pltpu.VMEM_SHARED((N_PAD, D), jnp.float32),
                   pltpu.SemaphoreType.DMA,
                   pltpu.SemaphoreType.DMA],
    compiler_params=_sc_params,
)


def _norms_body(hs_ref, hd_ref, ns_ref, nd_ref):
    ds_ = jnp.sum(hs_ref[...], axis=0)
    dd = jnp.sum(hd_ref[...], axis=0)
    ns_ref[...] = jnp.where(ds_ > 0, lax.rsqrt(jnp.maximum(ds_, 1.0)), 0.0)
    nd_ref[...] = jnp.where(dd > 0, lax.rsqrt(jnp.maximum(dd, 1.0)), 0.0)


_norms_call = pl.pallas_call(
    _norms_body,
    out_shape=[jax.ShapeDtypeStruct((N_PAD,), jnp.float32),
               jax.ShapeDtypeStruct((N_PAD,), jnp.float32)],
)


def _pre_body(h_ref, w_ref, ns_ref, xs_ref):
    x = jnp.dot(h_ref[...], w_ref[...], preferred_element_type=jnp.float32)
    xs_ref[0:N, :] = x * ns_ref[0:N, :]
    xs_ref[N:, :] = jnp.zeros((N_PAD - N, D), jnp.float32)


_pre_call = pl.pallas_call(
    _pre_body,
    out_shape=jax.ShapeDtypeStruct((N_PAD, D), jnp.float32),
)


def _post(p_ref, nd_ref, sn_ref, b_ref, g_ref, be_ref, h_prev):
    agg = p_ref[0, 0:N, :] + p_ref[1, 0:N, :]
    x = agg * nd_ref[0:N, :] + b_ref[...]
    x = x * sn_ref[...]
    mean = jnp.mean(x, axis=0)
    var = jnp.mean((x - mean) ** 2, axis=0)
    x = (x - mean) * lax.rsqrt(var + 1e-5) * g_ref[...] + be_ref[...]
    return h_prev + jnp.maximum(x, 0.0)


def _mid_body(p_ref, nd_ref, sn_ref, b_ref, g_ref, be_ref, h0_ref, ns_ref,
              w1_ref, xs1_ref, h1_ref):
    h1 = _post(p_ref, nd_ref, sn_ref, b_ref, g_ref, be_ref, h0_ref[...])
    h1_ref[...] = h1
    x1 = jnp.dot(h1, w1_ref[...], preferred_element_type=jnp.float32)
    xs1_ref[0:N, :] = x1 * ns_ref[0:N, :]
    xs1_ref[N:, :] = jnp.zeros((N_PAD - N, D), jnp.float32)


_mid_call = pl.pallas_call(
    _mid_body,
    out_shape=[jax.ShapeDtypeStruct((N_PAD, D), jnp.float32),
               jax.ShapeDtypeStruct((N, D), jnp.float32)],
)


def _final_body(p_ref, nd_ref, sn_ref, b_ref, g_ref, be_ref, h1_ref, hg_ref):
    h2 = _post(p_ref, nd_ref, sn_ref, b_ref, g_ref, be_ref, h1_ref[...])
    hg_ref[...] = jnp.mean(h2, axis=0, keepdims=True)


_final_call = pl.pallas_call(
    _final_body,
    out_shape=jax.ShapeDtypeStruct((1, D), jnp.float32),
)


def kernel(nodes_feat, edge_index, edges_feat, nodes_num_norm_sqrt,
           edges_num_norm_sqrt, W0, b0, gamma0, beta0, W1, b1, gamma1, beta1):
    del edges_feat, edges_num_norm_sqrt
    src = edge_index[0]
    dst = edge_index[1]
    padv = jnp.full((E_PAD - E,), N, jnp.int32)
    src1 = jnp.concatenate([src, padv])
    dst1 = jnp.concatenate([dst, padv])
    src2 = src1.reshape(NW * CH_PER_W, CHUNK)
    dst2 = dst1.reshape(NW * CH_PER_W, CHUNK)

    degs, degd = _degree_call(src1, dst1)
    ds_ = jnp.sum(degs, axis=0)
    dd = jnp.sum(degd, axis=0)
    ns = jnp.where(ds_ > 0, lax.rsqrt(jnp.maximum(ds_, 1.0)), 0.0)
    nd = jnp.where(dd > 0, lax.rsqrt(jnp.maximum(dd, 1.0)), 0.0)
    ns_col = ns.reshape(N_PAD, 1)
    nd_col = nd.reshape(N_PAD, 1)

    def post(part, b, g, be, h_prev):
        agg = part[0, :N] + part[1, :N]
        x = agg * nd_col[:N] + b
        x = x * nodes_num_norm_sqrt
        mean = jnp.mean(x, axis=0)
        var = jnp.mean((x - mean) ** 2, axis=0)
        x = (x - mean) * lax.rsqrt(var + 1e-5) * g + be
        return h_prev + jnp.maximum(x, 0.0)

    x0 = jnp.dot(nodes_feat, W0) * ns_col[:N]
    xs0 = jnp.concatenate([x0, jnp.zeros((N_PAD - N, D), jnp.float32)])
    part0 = _scatter_call(xs0, src2, dst2)
    h1 = post(part0, b0, gamma0, beta0, nodes_feat)
    x1 = jnp.dot(h1, W1) * ns_col[:N]
    xs1 = jnp.concatenate([x1, jnp.zeros((N_PAD - N, D), jnp.float32)])
    part1 = _scatter_call(xs1, src2, dst2)
    h2 = post(part1, b1, gamma1, beta1, h1)
    return jnp.mean(h2, axis=0, keepdims=True)
